# Initial kernel scaffold; baseline (speedup 1.0000x reference)
import jax
import jax.numpy as jnp
from jax.experimental import pallas as pl


def kernel(unit_means, labels, neighborhood_ids, neighborhood_explore_units):
    raise NotImplementedError("write your pallas kernel here")


# speedup vs baseline: 14.2215x; 14.2215x over previous
import functools

import jax
import jax.numpy as jnp
from jax import lax
from jax.experimental import pallas as pl
from jax.experimental.pallas import tpu as pltpu
from jax.experimental.pallas import tpu_sc as plsc

U = 2048
D = 256
S = 100000
NB = 512
EC = 64
NCAND = 10

NC = 2
NS = 16
SPT = 6400
SPAD = NS * SPT
BLK = 128
NBLK = SPT // BLK
CIDX = (U + 1) * NB
H = CIDX // NC
CSLICE = H // NS
STG = CSLICE // 6

_ROWB = 128


def _z():
    return jnp.int32(0)


def _dists_body(xb_ref, xf_ref, expl_ref, d_ref, nn_ref, ne_ref):
    xb = xb_ref[...]
    xf = xf_ref[...]
    g = lax.dot_general(
        xb, xf, (((1,), (1,)), ((), ())),
        preferred_element_type=jnp.float32,
        precision=lax.Precision.DEFAULT,
    )
    sqb = jnp.sum(xb * xb, axis=1, keepdims=True)
    sqf = jnp.sum(xf * xf, axis=1)[None, :]
    d = sqb + sqf - 2.0 * g
    d_ref[...] = d

    col = lax.broadcasted_iota(jnp.int32, d.shape, 1)
    big = jnp.float32(3.0e38)
    vals = d
    idxs = []
    for _ in range(3):
        m = jnp.min(vals, axis=1, keepdims=True)
        i = jnp.min(jnp.where(vals == m, col, jnp.int32(1 << 30)), axis=1)
        idxs.append(i)
        vals = jnp.where(col == i[:, None], big, vals)
    c2 = lax.broadcasted_iota(jnp.int32, (_ROWB, 128), 1)
    nn = jnp.where(
        c2 == 0, idxs[0][:, None],
        jnp.where(c2 == 1, idxs[1][:, None],
                  jnp.where(c2 == 2, idxs[2][:, None], jnp.int32(0))))
    nn_ref[...] = nn

    @pl.when(pl.program_id(0) == 0)
    def _():
        ne = jnp.sum((expl_ref[...] < U).astype(jnp.int32), axis=1,
                     dtype=jnp.int32)
        ne_ref[...] = jnp.broadcast_to(ne[None, :], (8, NB))


def _dists_topk(x, expl_i32):
    grid = (U // _ROWB,)
    return pl.pallas_call(
        _dists_body,
        grid=grid,
        in_specs=[
            pl.BlockSpec((_ROWB, D), lambda i: (i, _z())),
            pl.BlockSpec((U, D), lambda i: (_z(), _z())),
            pl.BlockSpec((NB, EC), lambda i: (_z(), _z())),
        ],
        out_specs=[
            pl.BlockSpec((_ROWB, U), lambda i: (i, _z())),
            pl.BlockSpec((_ROWB, 128), lambda i: (i, _z())),
            pl.BlockSpec((8, NB), lambda i: (_z(), _z())),
        ],
        out_shape=[
            jax.ShapeDtypeStruct((U, U), jnp.float32),
            jax.ShapeDtypeStruct((U, 128), jnp.int32),
            jax.ShapeDtypeStruct((8, NB), jnp.int32),
        ],
    )(x, x, expl_i32)


def _sc_body(pk_hbm, nn0_hbm, nn1_hbm, nn2_hbm, ne_hbm, expl_hbm,
             zero_hbm, cand_hbm, part_hbm,
             pk_v, nn0, nn1, nn2, ne_v, expl_v,
             idx0, idx1, cbuf0, cbuf1, ones_v, stage_v, counts_sh,
             semc0, semc1, sems0, sems1):
    i32 = jnp.int32
    cid = lax.axis_index("c")
    sid = lax.axis_index("s")
    base = sid * i32(SPT)
    lo = cid * i32(H)

    pltpu.sync_copy(pk_hbm.at[pl.ds(base, SPT)], pk_v)
    pltpu.sync_copy(nn0_hbm, nn0)
    pltpu.sync_copy(nn1_hbm, nn1)
    pltpu.sync_copy(nn2_hbm, nn2)
    pltpu.sync_copy(ne_hbm, ne_v)
    pltpu.sync_copy(expl_hbm, expl_v)
    pltpu.sync_copy(zero_hbm, stage_v)
    for k in range(6):
        pltpu.sync_copy(
            stage_v,
            counts_sh.at[pl.ds(sid * i32(CSLICE) + i32(k * STG), STG)])
    lanes16 = jnp.ones((16,), jnp.int32)
    for c in range(BLK // 16):
        ones_v[pl.ds(i32(c * 16), 16)] = lanes16
    plsc.subcore_barrier()

    lane = jnp.arange(16, dtype=jnp.int32)

    def do_block(b, idx_buf, cand_buf, semc, sems):
        sbase = b * i32(BLK)
        for c in range(BLK // 16):
            off = sbase + i32(c * 16)
            p = pk_v[pl.ds(off, 16)]
            lbl = p & i32(4095)
            nbv = lax.shift_right_logical(p, i32(12))
            t0 = plsc.load_gather(nn0, [lbl])
            t1 = plsc.load_gather(nn1, [lbl])
            t2 = plsc.load_gather(nn2, [lbl])
            s00 = plsc.load_gather(nn1, [t0])
            s01 = plsc.load_gather(nn2, [t0])
            s10 = plsc.load_gather(nn1, [t1])
            s11 = plsc.load_gather(nn2, [t1])
            s20 = plsc.load_gather(nn1, [t2])
            s21 = plsc.load_gather(nn2, [t2])
            ne16 = plsc.load_gather(ne_v, [nbv])
            targ = (lbl * i32(1000003) + i32(12345)) % \
                jnp.maximum(ne16, i32(1))
            ex = plsc.load_gather(expl_v, [nbv, targ])
            valid = (base + off + lane) < i32(S)
            cands = (t0, t1, t2, s00, s01, s10, s11, s20, s21, ex)
            for j, cv in enumerate(cands):
                cand_buf[i32(j), pl.ds(i32(c * 16), 16)] = cv
                local = cv * i32(NB) + nbv - lo
                inr = valid & (local >= i32(0)) & (local < i32(H))
                pad = i32(H) + (local & i32(4095))
                idx_buf[i32(j), pl.ds(i32(c * 16), 16)] = \
                    jnp.where(inr, local, pad)
        for j in range(NCAND):
            pltpu.async_copy(
                cand_buf.at[i32(j)],
                cand_hbm.at[pl.ds(i32(j * SPAD) + base + sbase, BLK)], semc)
            pltpu.async_copy(ones_v, counts_sh.at[idx_buf.at[i32(j)]], sems,
                             add=True)

    def drain(idx_buf, semc, sems):
        pltpu.make_async_copy(zero_hbm.at[pl.ds(i32(0), BLK * NCAND)],
                              stage_v.at[pl.ds(i32(0), BLK * NCAND)],
                              semc).wait()
        for j in range(NCAND):
            pltpu.make_async_copy(ones_v, counts_sh.at[idx_buf.at[i32(j)]],
                                  sems).wait()

    do_block(i32(0), idx0, cbuf0, semc0, sems0)
    do_block(i32(1), idx1, cbuf1, semc1, sems1)

    def body(k, carry):
        b = i32(2) * k + i32(2)
        drain(idx0, semc0, sems0)
        do_block(b, idx0, cbuf0, semc0, sems0)
        drain(idx1, semc1, sems1)
        do_block(b + i32(1), idx1, cbuf1, semc1, sems1)
        return carry

    lax.fori_loop(i32(0), i32((NBLK - 2) // 2), body, i32(0))
    drain(idx0, semc0, sems0)
    drain(idx1, semc1, sems1)
    plsc.subcore_barrier()
    for k in range(6):
        coff = sid * i32(CSLICE) + i32(k * STG)
        pltpu.sync_copy(counts_sh.at[pl.ds(coff, STG)], stage_v)
        pltpu.sync_copy(stage_v, part_hbm.at[pl.ds(lo + coff, STG)])


def _sc_call(pk, nn0, nn1, nn2, ne, expl_i32, zeros):
    mesh = plsc.VectorSubcoreMesh(core_axis_name="c", subcore_axis_name="s",
                                  num_cores=NC, num_subcores=NS)
    f = functools.partial(
        pl.kernel,
        out_type=[
            jax.ShapeDtypeStruct((SPAD * NCAND,), jnp.int32),
            jax.ShapeDtypeStruct((CIDX,), jnp.int32),
        ],
        mesh=mesh,
        compiler_params=pltpu.CompilerParams(needs_layout_passes=False),
        scratch_types=[
            pltpu.VMEM((SPT,), jnp.int32),
            pltpu.VMEM((U,), jnp.int32),
            pltpu.VMEM((U,), jnp.int32),
            pltpu.VMEM((U,), jnp.int32),
            pltpu.VMEM((NB,), jnp.int32),
            pltpu.VMEM((NB, EC), jnp.int32),
            pltpu.VMEM((NCAND, BLK), jnp.int32),
            pltpu.VMEM((NCAND, BLK), jnp.int32),
            pltpu.VMEM((NCAND, BLK), jnp.int32),
            pltpu.VMEM((NCAND, BLK), jnp.int32),
            pltpu.VMEM((BLK,), jnp.int32),
            pltpu.VMEM((STG,), jnp.int32),
            pltpu.VMEM_SHARED((H + 4096,), jnp.int32),
            pltpu.SemaphoreType.DMA,
            pltpu.SemaphoreType.DMA,
            pltpu.SemaphoreType.DMA,
            pltpu.SemaphoreType.DMA,
        ],
    )(_sc_body)
    return f(pk, nn0, nn1, nn2, ne, expl_i32, zeros)


def kernel(unit_means, labels, neighborhood_ids, neighborhood_explore_units):
    expl_i32 = neighborhood_explore_units.astype(jnp.int32)
    dists, nnpad, ne8 = _dists_topk(unit_means, expl_i32)
    nn0 = nnpad[:, 0]
    nn1 = nnpad[:, 1]
    nn2 = nnpad[:, 2]
    ne = ne8[0]

    pk_s = (labels.astype(jnp.int32) |
            (neighborhood_ids.astype(jnp.int32) << 12))
    pk = jnp.zeros((SPAD,), jnp.int32).at[:S].set(pk_s)
    zeros = jnp.zeros((STG,), jnp.int32)

    cand_flat, counts_flat = _sc_call(pk, nn0, nn1, nn2, ne,
                                      expl_i32, zeros)
    counts = counts_flat.reshape(U + 1, NB)
    cand_t = cand_flat.reshape(NCAND, SPAD)
    candidates = cand_t[:, :S].T.astype(jnp.int64)
    return candidates, counts, dists


# speedup vs baseline: 17.0211x; 1.1969x over previous
import functools

import jax
import jax.numpy as jnp
from jax import lax
from jax.experimental import pallas as pl
from jax.experimental.pallas import tpu as pltpu
from jax.experimental.pallas import tpu_sc as plsc

U = 2048
D = 256
S = 100000
NB = 512
EC = 64
NCAND = 10

NC = 2
NS = 16
SPT = 6400
SPAD = NS * SPT
BLK = 128
NBLK = SPT // BLK
CIDX = (U + 1) * NB
H = CIDX // NC
CSLICE = H // NS
STG = CSLICE // 6

_ROWB = 128


def _z():
    return jnp.int32(0)


def _dists_body(xb_ref, xf_ref, expl_ref, d_ref, nn_ref, ne_ref):
    xb = xb_ref[...]
    xf = xf_ref[...]
    g = lax.dot_general(
        xb, xf, (((1,), (1,)), ((), ())),
        preferred_element_type=jnp.float32,
        precision=lax.Precision.DEFAULT,
    )
    sqb = jnp.sum(xb * xb, axis=1, keepdims=True)
    sqf = jnp.sum(xf * xf, axis=1)[None, :]
    d = sqb + sqf - 2.0 * g
    d_ref[...] = d

    col = lax.broadcasted_iota(jnp.int32, d.shape, 1)
    big = jnp.float32(3.0e38)
    vals = d
    idxs = []
    for _ in range(3):
        m = jnp.min(vals, axis=1, keepdims=True)
        i = jnp.min(jnp.where(vals == m, col, jnp.int32(1 << 30)), axis=1)
        idxs.append(i)
        vals = jnp.where(col == i[:, None], big, vals)
    c2 = lax.broadcasted_iota(jnp.int32, (_ROWB, 128), 1)
    nn = jnp.where(
        c2 == 0, idxs[0][:, None],
        jnp.where(c2 == 1, idxs[1][:, None],
                  jnp.where(c2 == 2, idxs[2][:, None], jnp.int32(0))))
    nn_ref[...] = nn

    @pl.when(pl.program_id(0) == 0)
    def _():
        ne = jnp.sum((expl_ref[...] < U).astype(jnp.int32), axis=1,
                     dtype=jnp.int32)
        ne_ref[...] = jnp.broadcast_to(ne[None, :], (8, NB))


def _dists_topk(x, expl_i32):
    grid = (U // _ROWB,)
    return pl.pallas_call(
        _dists_body,
        grid=grid,
        in_specs=[
            pl.BlockSpec((_ROWB, D), lambda i: (i, _z())),
            pl.BlockSpec((U, D), lambda i: (_z(), _z())),
            pl.BlockSpec((NB, EC), lambda i: (_z(), _z())),
        ],
        out_specs=[
            pl.BlockSpec((_ROWB, U), lambda i: (i, _z())),
            pl.BlockSpec((_ROWB, 128), lambda i: (i, _z())),
            pl.BlockSpec((8, NB), lambda i: (_z(), _z())),
        ],
        out_shape=[
            jax.ShapeDtypeStruct((U, U), jnp.float32),
            jax.ShapeDtypeStruct((U, 128), jnp.int32),
            jax.ShapeDtypeStruct((8, NB), jnp.int32),
        ],
    )(x, x, expl_i32)


SPT_A = SPAD // (NC * NS)
NBLK_A = SPT_A // BLK


def _cand_body(pk_hbm, nn0_hbm, nn1_hbm, nn2_hbm, ne_hbm, expl_hbm,
               cand_hbm,
               pk_v, nn0, nn1, nn2, ne_v, expl_v, cbuf0, cbuf1,
               sem0, sem1):
    i32 = jnp.int32
    cid = lax.axis_index("c")
    sid = lax.axis_index("s")
    wid = cid * i32(NS) + sid
    base = wid * i32(SPT_A)

    pltpu.sync_copy(pk_hbm.at[pl.ds(base, SPT_A)], pk_v)
    pltpu.sync_copy(nn0_hbm, nn0)
    pltpu.sync_copy(nn1_hbm, nn1)
    pltpu.sync_copy(nn2_hbm, nn2)
    pltpu.sync_copy(ne_hbm, ne_v)
    pltpu.sync_copy(expl_hbm, expl_v)

    def do_block(b, cand_buf, sem):
        sbase = b * i32(BLK)
        for c in range(BLK // 16):
            off = sbase + i32(c * 16)
            p = pk_v[pl.ds(off, 16)]
            lbl = p & i32(4095)
            nbv = lax.shift_right_logical(p, i32(12))
            t0 = plsc.load_gather(nn0, [lbl])
            t1 = plsc.load_gather(nn1, [lbl])
            t2 = plsc.load_gather(nn2, [lbl])
            s00 = plsc.load_gather(nn1, [t0])
            s01 = plsc.load_gather(nn2, [t0])
            s10 = plsc.load_gather(nn1, [t1])
            s11 = plsc.load_gather(nn2, [t1])
            s20 = plsc.load_gather(nn1, [t2])
            s21 = plsc.load_gather(nn2, [t2])
            ne16 = plsc.load_gather(ne_v, [nbv])
            targ = (lbl * i32(1000003) + i32(12345)) % \
                jnp.maximum(ne16, i32(1))
            ex = plsc.load_gather(expl_v, [nbv, targ])
            cands = (t0, t1, t2, s00, s01, s10, s11, s20, s21, ex)
            for j, cv in enumerate(cands):
                cand_buf[i32(j), pl.ds(i32(c * 16), 16)] = cv
        for j in range(NCAND):
            pltpu.async_copy(
                cand_buf.at[i32(j)],
                cand_hbm.at[pl.ds(i32(j * SPAD) + base + sbase, BLK)], sem)

    def drain(sem):
        pltpu.make_async_copy(pk_hbm.at[pl.ds(i32(0), BLK * NCAND)],
                              pk_v.at[pl.ds(i32(0), BLK * NCAND)],
                              sem).wait()

    do_block(i32(0), cbuf0, sem0)
    do_block(i32(1), cbuf1, sem1)

    def body(k, carry):
        b = i32(2) * k + i32(2)
        drain(sem0)
        do_block(b, cbuf0, sem0)
        drain(sem1)
        do_block(b + i32(1), cbuf1, sem1)
        return carry

    lax.fori_loop(i32(0), i32((NBLK_A - 2) // 2), body, i32(0))
    drain(sem0)
    drain(sem1)


def _cand_call(pk, nn0, nn1, nn2, ne, expl_i32):
    mesh = plsc.VectorSubcoreMesh(core_axis_name="c", subcore_axis_name="s",
                                  num_cores=NC, num_subcores=NS)
    f = functools.partial(
        pl.kernel,
        out_type=jax.ShapeDtypeStruct((SPAD * NCAND,), jnp.int32),
        mesh=mesh,
        compiler_params=pltpu.CompilerParams(needs_layout_passes=False),
        scratch_types=[
            pltpu.VMEM((SPT_A,), jnp.int32),
            pltpu.VMEM((U,), jnp.int32),
            pltpu.VMEM((U,), jnp.int32),
            pltpu.VMEM((U,), jnp.int32),
            pltpu.VMEM((NB,), jnp.int32),
            pltpu.VMEM((NB, EC), jnp.int32),
            pltpu.VMEM((NCAND, BLK), jnp.int32),
            pltpu.VMEM((NCAND, BLK), jnp.int32),
            pltpu.SemaphoreType.DMA,
            pltpu.SemaphoreType.DMA,
        ],
    )(_cand_body)
    return f(pk, nn0, nn1, nn2, ne, expl_i32)


def _sc_body(pk_hbm, nn0_hbm, nn1_hbm, nn2_hbm, ne_hbm, expl_hbm,
             zero_hbm, part_hbm,
             pk_v, nn0, nn1, nn2, ne_v, expl_v,
             idx0, idx1, ones_v, stage_v, counts_sh,
             sems0, sems1):
    i32 = jnp.int32
    cid = lax.axis_index("c")
    sid = lax.axis_index("s")
    base = sid * i32(SPT)
    lo = cid * i32(H)

    pltpu.sync_copy(pk_hbm.at[pl.ds(base, SPT)], pk_v)
    pltpu.sync_copy(nn0_hbm, nn0)
    pltpu.sync_copy(nn1_hbm, nn1)
    pltpu.sync_copy(nn2_hbm, nn2)
    pltpu.sync_copy(ne_hbm, ne_v)
    pltpu.sync_copy(expl_hbm, expl_v)
    pltpu.sync_copy(zero_hbm, stage_v)
    for k in range(6):
        pltpu.sync_copy(
            stage_v,
            counts_sh.at[pl.ds(sid * i32(CSLICE) + i32(k * STG), STG)])
    lanes16 = jnp.ones((16,), jnp.int32)
    for c in range(BLK // 16):
        ones_v[pl.ds(i32(c * 16), 16)] = lanes16
    plsc.subcore_barrier()

    lane = jnp.arange(16, dtype=jnp.int32)

    def do_block(b, idx_buf, sems):
        sbase = b * i32(BLK)
        for c in range(BLK // 16):
            off = sbase + i32(c * 16)
            p = pk_v[pl.ds(off, 16)]
            lbl = p & i32(4095)
            nbv = lax.shift_right_logical(p, i32(12))
            t0 = plsc.load_gather(nn0, [lbl])
            t1 = plsc.load_gather(nn1, [lbl])
            t2 = plsc.load_gather(nn2, [lbl])
            s00 = plsc.load_gather(nn1, [t0])
            s01 = plsc.load_gather(nn2, [t0])
            s10 = plsc.load_gather(nn1, [t1])
            s11 = plsc.load_gather(nn2, [t1])
            s20 = plsc.load_gather(nn1, [t2])
            s21 = plsc.load_gather(nn2, [t2])
            ne16 = plsc.load_gather(ne_v, [nbv])
            targ = (lbl * i32(1000003) + i32(12345)) % \
                jnp.maximum(ne16, i32(1))
            ex = plsc.load_gather(expl_v, [nbv, targ])
            valid = (base + off + lane) < i32(S)
            cands = (t0, t1, t2, s00, s01, s10, s11, s20, s21, ex)
            for j, cv in enumerate(cands):
                local = cv * i32(NB) + nbv - lo
                inr = valid & (local >= i32(0)) & (local < i32(H))
                pad = i32(H) + (local & i32(4095))
                idx_buf[i32(j), pl.ds(i32(c * 16), 16)] = \
                    jnp.where(inr, local, pad)
        for j in range(NCAND):
            pltpu.async_copy(ones_v, counts_sh.at[idx_buf.at[i32(j)]], sems,
                             add=True)

    def drain(idx_buf, sems):
        for j in range(NCAND):
            pltpu.make_async_copy(ones_v, counts_sh.at[idx_buf.at[i32(j)]],
                                  sems).wait()

    do_block(i32(0), idx0, sems0)
    do_block(i32(1), idx1, sems1)

    def body(k, carry):
        b = i32(2) * k + i32(2)
        drain(idx0, sems0)
        do_block(b, idx0, sems0)
        drain(idx1, sems1)
        do_block(b + i32(1), idx1, sems1)
        return carry

    lax.fori_loop(i32(0), i32((NBLK - 2) // 2), body, i32(0))
    drain(idx0, sems0)
    drain(idx1, sems1)
    plsc.subcore_barrier()
    for k in range(6):
        coff = sid * i32(CSLICE) + i32(k * STG)
        pltpu.sync_copy(counts_sh.at[pl.ds(coff, STG)], stage_v)
        pltpu.sync_copy(stage_v, part_hbm.at[pl.ds(lo + coff, STG)])


def _sc_call(pk, nn0, nn1, nn2, ne, expl_i32, zeros):
    mesh = plsc.VectorSubcoreMesh(core_axis_name="c", subcore_axis_name="s",
                                  num_cores=NC, num_subcores=NS)
    f = functools.partial(
        pl.kernel,
        out_type=jax.ShapeDtypeStruct((CIDX,), jnp.int32),
        mesh=mesh,
        compiler_params=pltpu.CompilerParams(needs_layout_passes=False),
        scratch_types=[
            pltpu.VMEM((SPT,), jnp.int32),
            pltpu.VMEM((U,), jnp.int32),
            pltpu.VMEM((U,), jnp.int32),
            pltpu.VMEM((U,), jnp.int32),
            pltpu.VMEM((NB,), jnp.int32),
            pltpu.VMEM((NB, EC), jnp.int32),
            pltpu.VMEM((NCAND, BLK), jnp.int32),
            pltpu.VMEM((NCAND, BLK), jnp.int32),
            pltpu.VMEM((BLK,), jnp.int32),
            pltpu.VMEM((STG,), jnp.int32),
            pltpu.VMEM_SHARED((H + 4096,), jnp.int32),
            pltpu.SemaphoreType.DMA,
            pltpu.SemaphoreType.DMA,
        ],
    )(_sc_body)
    return f(pk, nn0, nn1, nn2, ne, expl_i32, zeros)


def kernel(unit_means, labels, neighborhood_ids, neighborhood_explore_units):
    expl_i32 = neighborhood_explore_units.astype(jnp.int32)
    dists, nnpad, ne8 = _dists_topk(unit_means, expl_i32)
    nn0 = nnpad[:, 0]
    nn1 = nnpad[:, 1]
    nn2 = nnpad[:, 2]
    ne = ne8[0]

    pk_s = (labels.astype(jnp.int32) |
            (neighborhood_ids.astype(jnp.int32) << 12))
    pk = jnp.zeros((SPAD,), jnp.int32).at[:S].set(pk_s)
    zeros = jnp.zeros((STG,), jnp.int32)

    cand_flat = _cand_call(pk, nn0, nn1, nn2, ne, expl_i32)
    zeros_dep = zeros + cand_flat[0] * jnp.int32(0)
    counts_flat = _sc_call(pk, nn0, nn1, nn2, ne, expl_i32, zeros_dep)
    counts = counts_flat.reshape(U + 1, NB)
    cand_t = cand_flat.reshape(NCAND, SPAD)
    candidates = cand_t[:, :S].T.astype(jnp.int64)
    return candidates, counts, dists
